# Initial kernel scaffold; baseline (speedup 1.0000x reference)
import jax
import jax.numpy as jnp
from jax.experimental import pallas as pl


def kernel(x, target):
    raise NotImplementedError("write your pallas kernel here")


# speedup vs baseline: 2.2221x; 2.2221x over previous
import functools
import math

import jax
import jax.numpy as jnp
from jax import lax
from jax.experimental import pallas as pl
from jax.experimental.pallas import tpu as pltpu
from jax.experimental.pallas import tpu_sc as plsc

_PAD = 0
_SMOOTHING = 0.1
_CONF = 1.0 - _SMOOTHING

_NC, _NS = 2, 16
_NW = _NC * _NS
_LANES = 16


def _dense_body(t_ref, x_ref, out_ref):
    j = pl.program_id(0)
    blk = x_ref[...]
    m = (t_ref[...] != _PAD).astype(jnp.float32)

    @pl.when(j == 0)
    def _init():
        out_ref[0] = jnp.sum(blk[:, 0:1] * m)
        out_ref[1] = jnp.sum(m)
        out_ref[2] = 0.0

    out_ref[2] += jnp.sum(blk * m)


def _dense_stats(x, t2d):
    N, V = x.shape
    bc = 1280
    return pl.pallas_call(
        _dense_body,
        grid=(V // bc,),
        in_specs=[
            pl.BlockSpec((N, 1), lambda j: (0, 0)),
            pl.BlockSpec((N, bc), lambda j: (0, j)),
        ],
        out_specs=pl.BlockSpec(memory_space=pltpu.SMEM),
        out_shape=jax.ShapeDtypeStruct((3,), jnp.float32),
    )(t2d, x)


def _sc_gather_sum(x_flat, t, v):
    n = t.shape[0]
    rpw = n // _NW
    mesh = plsc.VectorSubcoreMesh(core_axis_name="c", subcore_axis_name="s")

    @functools.partial(
        pl.kernel,
        mesh=mesh,
        out_type=jax.ShapeDtypeStruct((_NW, _LANES), jnp.float32),
        scratch_types=[
            pltpu.VMEM((rpw,), jnp.int32),
            pltpu.VMEM((rpw,), jnp.int32),
            pltpu.VMEM((rpw,), jnp.float32),
            pltpu.VMEM((_LANES,), jnp.float32),
            pltpu.SemaphoreType.DMA,
        ],
    )
    def k(xf_hbm, t_hbm, out_hbm, t_v, idx_v, val_v, acc_v, sem):
        wid = lax.axis_index("s") * _NC + lax.axis_index("c")
        base = wid * rpw
        pltpu.sync_copy(t_hbm.at[pl.ds(base, rpw)], t_v)
        for kk in range(rpw // _LANES):
            rows = base + kk * _LANES + lax.iota(jnp.int32, _LANES)
            idx_v[pl.ds(kk * _LANES, _LANES)] = (
                rows * v + t_v[pl.ds(kk * _LANES, _LANES)]
            )
        pltpu.async_copy(xf_hbm.at[idx_v], val_v, sem).wait()
        acc = jnp.zeros((_LANES,), jnp.float32)
        for kk in range(rpw // _LANES):
            tt = t_v[pl.ds(kk * _LANES, _LANES)]
            vv = val_v[pl.ds(kk * _LANES, _LANES)]
            acc = acc + jnp.where(tt != _PAD, vv, 0.0)
        acc_v[...] = acc
        pltpu.sync_copy(acc_v, out_hbm.at[wid])

    return k(x_flat, t)


def kernel(x, target):
    n, v = x.shape
    t32 = target.astype(jnp.int32)
    stats = _dense_stats(x, t32.reshape(n, 1))
    parts = _sc_gather_sum(x.reshape(-1), t32, v)
    a = jnp.sum(parts)
    eps = _SMOOTHING / (v - 2)
    c = _CONF * math.log(_CONF) + (v - 2) * eps * math.log(eps)
    b, cnt, s = stats[0], stats[1], stats[2]
    return cnt * c - (_CONF - eps) * a - eps * (s - b)


# speedup vs baseline: 5.6687x; 2.5511x over previous
import math

import jax
import jax.numpy as jnp
from jax import lax
from jax.experimental import pallas as pl
from jax.experimental.pallas import tpu as pltpu

_PAD = 0
_SMOOTHING = 0.1
_CONF = 1.0 - _SMOOTHING


def _dense_body(t_ref, x_ref, out_ref, *, bc):
    j = pl.program_id(0)
    blk = x_ref[...]
    t = t_ref[...]
    m = (t != _PAD).astype(jnp.float32)
    n = blk.shape[0]
    cols = j * bc + lax.broadcasted_iota(jnp.int32, (n, bc), 1)
    xt = jnp.where(cols == t, blk, 0.0)

    @pl.when(j == 0)
    def _init():
        out_ref[0] = jnp.sum(blk[:, 0:1] * m)
        out_ref[1] = jnp.sum(m)
        out_ref[2] = 0.0
        out_ref[3] = 0.0

    out_ref[2] += jnp.sum(blk * m)
    out_ref[3] += jnp.sum(xt * m)


def _dense_stats(x, t2d):
    import functools
    N, V = x.shape
    bc = 1280
    return pl.pallas_call(
        functools.partial(_dense_body, bc=bc),
        grid=(V // bc,),
        in_specs=[
            pl.BlockSpec((N, 1), lambda j: (0, 0)),
            pl.BlockSpec((N, bc), lambda j: (0, j)),
        ],
        out_specs=pl.BlockSpec(memory_space=pltpu.SMEM),
        out_shape=jax.ShapeDtypeStruct((4,), jnp.float32),
    )(t2d, x)


def kernel(x, target):
    n, v = x.shape
    t32 = target.astype(jnp.int32)
    stats = _dense_stats(x, t32.reshape(n, 1))
    eps = _SMOOTHING / (v - 2)
    c = _CONF * math.log(_CONF) + (v - 2) * eps * math.log(eps)
    b, cnt, s, a = stats[0], stats[1], stats[2], stats[3]
    return cnt * c - (_CONF - eps) * a - eps * (s - b)
